# baseline (speedup 1.0000x reference)
import jax
import jax.numpy as jnp
from jax import lax
from jax.experimental import pallas as pl
from jax.experimental.pallas import tpu as pltpu
from jax.experimental.pallas import tpu_sc as plsc

B, C, H, W = 4, 8, 1024, 1024
N_ERA = 65536
N = H * W
NCH = B * C
LANES = 16
CHUNK = 4096
NCHUNK = N // 2 // CHUNK
ROWS = CHUNK // W
HROW = H // 2
SEG_SCALE = float(N_ERA) / float(N)
NC, NS = 2, 16
NBUF = 4


def _sc_body(data_hbm, map_hbm, out_hbm, *refs):
    idxs = refs[0:NBUF]
    vlos = refs[NBUF:2 * NBUF]
    vhis = refs[2 * NBUF:3 * NBUF]
    acc = refs[3 * NBUF]
    sems_i = refs[3 * NBUF + 1:3 * NBUF + 1 + NBUF]
    sems_l = refs[3 * NBUF + 1 + NBUF:3 * NBUF + 1 + 2 * NBUF]
    sems_h = refs[3 * NBUF + 1 + 2 * NBUF:3 * NBUF + 1 + 3 * NBUF]
    wid = lax.axis_index("s") * NC + lax.axis_index("c")

    def fill(g, b):
        gg = lax.rem(g + wid * (NCHUNK // (NC * NS)), NCHUNK)
        pltpu.async_copy(map_hbm.at[pl.ds(gg * CHUNK, CHUNK)], idxs[b], sems_i[b])
        pltpu.async_copy(
            data_hbm.at[wid, pl.ds(gg * ROWS, ROWS), :], vlos[b], sems_l[b])
        pltpu.async_copy(
            data_hbm.at[wid, pl.ds(HROW + gg * ROWS, ROWS), :], vhis[b], sems_h[b])

    def wait(b):
        pltpu.make_async_copy(
            map_hbm.at[pl.ds(0, CHUNK)], idxs[b], sems_i[b]).wait()
        pltpu.make_async_copy(
            data_hbm.at[0, pl.ds(0, ROWS), :], vlos[b], sems_l[b]).wait()
        pltpu.make_async_copy(
            data_hbm.at[0, pl.ds(0, ROWS), :], vhis[b], sems_h[b]).wait()

    for b in range(NBUF):
        fill(b, b)

    zeros = jnp.zeros((LANES,), jnp.float32)

    def zero_body(i, carry):
        acc[pl.ds(i * LANES, LANES)] = zeros
        return carry

    lax.fori_loop(0, N_ERA // LANES, zero_body, 0, unroll=8)

    def scatter_chunk(b):
        idx_buf, vlo_buf, vhi_buf = idxs[b], vlos[b], vhis[b]
        @plsc.parallel_loop(0, CHUNK // LANES, unroll=8)
        def _(j):
            r = j // (W // LANES)
            c = (j % (W // LANES)) * LANES
            packed = idx_buf[pl.ds(j * LANES, LANES)]
            ia = packed & 0xFFFF
            ib = lax.shift_right_logical(packed, 16)
            va = vlo_buf[r, pl.ds(c, LANES)] * SEG_SCALE
            vb = vhi_buf[r, pl.ds(c, LANES)] * SEG_SCALE
            plsc.addupdate_scatter(acc, [ia], va)
            plsc.addupdate_scatter(acc, [ib], vb)

    def chunk_group(gp, carry):
        g0 = gp * NBUF
        for b in range(NBUF):
            wait(b)
            scatter_chunk(b)

            @pl.when(g0 + b + NBUF < NCHUNK)
            def _():
                fill(g0 + b + NBUF, b)

        return carry

    lax.fori_loop(0, NCHUNK // NBUF, chunk_group, 0)

    pltpu.sync_copy(acc, out_hbm.at[pl.ds(wid * N_ERA, N_ERA)])


@jax.jit
def _mean_to_era5(data3, packed_map):
    mesh = plsc.VectorSubcoreMesh(
        core_axis_name="c", subcore_axis_name="s", num_cores=NC, num_subcores=NS)
    return pl.kernel(
        _sc_body,
        out_type=jax.ShapeDtypeStruct((NCH * N_ERA,), jnp.float32),
        mesh=mesh,
        compiler_params=pltpu.CompilerParams(needs_layout_passes=False),
        scratch_types=(
            [pltpu.VMEM((CHUNK,), jnp.int32)] * NBUF
            + [pltpu.VMEM((ROWS, W), jnp.float32)] * (2 * NBUF)
            + [pltpu.VMEM((N_ERA,), jnp.float32)]
            + [pltpu.SemaphoreType.DMA] * (3 * NBUF)
        ),
    )(data3, packed_map)


def kernel(output, mapping):
    data3 = output.reshape(NCH, H, W)
    m32 = mapping[: N // 2] | (mapping[N // 2:] << 16)
    out_flat = _mean_to_era5(data3, m32)
    return out_flat.reshape(B, C, N_ERA)


# speedup vs baseline: 1.0064x; 1.0064x over previous
import jax
import jax.numpy as jnp
from jax import lax
from jax.experimental import pallas as pl
from jax.experimental.pallas import tpu as pltpu
from jax.experimental.pallas import tpu_sc as plsc

B, C, H, W = 4, 8, 1024, 1024
N_ERA = 65536
N = H * W
NCH = B * C
LANES = 16
CHUNK = 8192
NCHUNK = N // 2 // CHUNK
ROWS = CHUNK // W
HROW = H // 2
SEG_SCALE = float(N_ERA) / float(N)
NC, NS = 2, 16
NBUF = 2


def _sc_body(data_hbm, map_hbm, out_hbm, *refs):
    idxs = refs[0:NBUF]
    vlos = refs[NBUF:2 * NBUF]
    vhis = refs[2 * NBUF:3 * NBUF]
    acc = refs[3 * NBUF]
    sems_i = refs[3 * NBUF + 1:3 * NBUF + 1 + NBUF]
    sems_l = refs[3 * NBUF + 1 + NBUF:3 * NBUF + 1 + 2 * NBUF]
    sems_h = refs[3 * NBUF + 1 + 2 * NBUF:3 * NBUF + 1 + 3 * NBUF]
    wid = lax.axis_index("s") * NC + lax.axis_index("c")

    def fill(g, b):
        gg = lax.rem(g + wid * (NCHUNK // (NC * NS)), NCHUNK)
        pltpu.async_copy(map_hbm.at[pl.ds(gg * CHUNK, CHUNK)], idxs[b], sems_i[b])
        pltpu.async_copy(
            data_hbm.at[wid, pl.ds(gg * ROWS, ROWS), :], vlos[b], sems_l[b])
        pltpu.async_copy(
            data_hbm.at[wid, pl.ds(HROW + gg * ROWS, ROWS), :], vhis[b], sems_h[b])

    def wait(b):
        pltpu.make_async_copy(
            map_hbm.at[pl.ds(0, CHUNK)], idxs[b], sems_i[b]).wait()
        pltpu.make_async_copy(
            data_hbm.at[0, pl.ds(0, ROWS), :], vlos[b], sems_l[b]).wait()
        pltpu.make_async_copy(
            data_hbm.at[0, pl.ds(0, ROWS), :], vhis[b], sems_h[b]).wait()

    for b in range(NBUF):
        fill(b, b)

    zeros = jnp.zeros((LANES,), jnp.float32)

    def zero_body(i, carry):
        acc[pl.ds(i * LANES, LANES)] = zeros
        return carry

    lax.fori_loop(0, N_ERA // LANES, zero_body, 0, unroll=8)

    def scatter_chunk(b):
        idx_buf, vlo_buf, vhi_buf = idxs[b], vlos[b], vhis[b]
        @plsc.parallel_loop(0, CHUNK // LANES, unroll=8)
        def _(j):
            r = j // (W // LANES)
            c = (j % (W // LANES)) * LANES
            packed = idx_buf[pl.ds(j * LANES, LANES)]
            ia = packed & 0xFFFF
            ib = lax.shift_right_logical(packed, 16)
            va = vlo_buf[r, pl.ds(c, LANES)] * SEG_SCALE
            vb = vhi_buf[r, pl.ds(c, LANES)] * SEG_SCALE
            plsc.addupdate_scatter(acc, [ia], va)
            plsc.addupdate_scatter(acc, [ib], vb)

    def chunk_group(gp, carry):
        g0 = gp * NBUF
        for b in range(NBUF):
            wait(b)
            scatter_chunk(b)

            @pl.when(g0 + b + NBUF < NCHUNK)
            def _():
                fill(g0 + b + NBUF, b)

        return carry

    lax.fori_loop(0, NCHUNK // NBUF, chunk_group, 0)

    pltpu.sync_copy(acc, out_hbm.at[pl.ds(wid * N_ERA, N_ERA)])


@jax.jit
def _mean_to_era5(data3, packed_map):
    mesh = plsc.VectorSubcoreMesh(
        core_axis_name="c", subcore_axis_name="s", num_cores=NC, num_subcores=NS)
    return pl.kernel(
        _sc_body,
        out_type=jax.ShapeDtypeStruct((NCH * N_ERA,), jnp.float32),
        mesh=mesh,
        compiler_params=pltpu.CompilerParams(needs_layout_passes=False),
        scratch_types=(
            [pltpu.VMEM((CHUNK,), jnp.int32)] * NBUF
            + [pltpu.VMEM((ROWS, W), jnp.float32)] * (2 * NBUF)
            + [pltpu.VMEM((N_ERA,), jnp.float32)]
            + [pltpu.SemaphoreType.DMA] * (3 * NBUF)
        ),
    )(data3, packed_map)


def kernel(output, mapping):
    data3 = output.reshape(NCH, H, W)
    m32 = mapping[: N // 2] | (mapping[N // 2:] << 16)
    out_flat = _mean_to_era5(data3, m32)
    return out_flat.reshape(B, C, N_ERA)
